# baseline (speedup 1.0000x reference)
import functools

import jax
import jax.numpy as jnp
import numpy as np
from jax import lax
from jax.experimental import pallas as pl
from jax.experimental.pallas import tpu as pltpu
from jax.experimental.pallas import tpu_sc as plsc

N_ROWS = 64 * 52 * 52
N_WORKERS = 32
ROWS_PW = N_ROWS // N_WORKERS
FLOATS_PW = ROWS_PW * 5
CHUNKS = ROWS_PW // 16

_CELL = np.float32(416.0 / 52.0)
_HALF_IMG = np.float32(416.0 / 2)


def _softplus_neg(t):
    u = jnp.exp(-t)
    s = u / (np.float32(2.0) + u)
    s2 = s * s
    p = np.float32(1.0 / 11.0)
    for c in (1.0 / 9.0, 1.0 / 7.0, 1.0 / 5.0, 1.0 / 3.0, 1.0):
        p = p * s2 + np.float32(c)
    return np.float32(2.0) * s * p


def _sigmoid(x):
    return np.float32(1.0) / (np.float32(1.0) + jnp.exp(-x))


def _corners(xc, yc, w, h):
    cx = xc * _CELL
    cy = yc * _CELL
    hw = w * _HALF_IMG
    hh = h * _HALF_IMG
    return cx - hw, cy - hh, cx + hw, cy + hh


def _yolo_body(in_hbm, tg_hbm, out_hbm, in_buf, tg_buf, ob):
    wid = lax.axis_index("s") * 2 + lax.axis_index("c")
    base = wid * FLOATS_PW
    pltpu.sync_copy(in_hbm.at[pl.ds(base, FLOATS_PW)], in_buf)
    pltpu.sync_copy(tg_hbm.at[pl.ds(base, FLOATS_PW)], tg_buf)

    row_off = lax.iota(jnp.int32, 16) * 5

    def chunk(i, acc):
        a_noobj, a_obj, a_cnt, a_giou = acc
        off = i * 80 + row_off
        x = plsc.load_gather(in_buf, [off])
        conf = plsc.load_gather(tg_buf, [off])
        px = plsc.load_gather(in_buf, [off + 1])
        py = plsc.load_gather(in_buf, [off + 2])
        pw = plsc.load_gather(in_buf, [off + 3])
        ph = plsc.load_gather(in_buf, [off + 4])
        tx = plsc.load_gather(tg_buf, [off + 1])
        ty = plsc.load_gather(tg_buf, [off + 2])
        tw = plsc.load_gather(tg_buf, [off + 3])
        th = plsc.load_gather(tg_buf, [off + 4])

        relu = jnp.maximum(x, np.float32(0.0))
        sp = _softplus_neg(jnp.abs(x))
        bce = relu + sp
        noobjf = np.float32(1.0) - conf
        a_noobj = a_noobj + bce * noobjf
        a_obj = a_obj + (bce - x * conf) * conf
        a_cnt = a_cnt + conf

        ax0, ay0, ax1, ay1 = _corners(
            _sigmoid(px), _sigmoid(py), _sigmoid(pw), _sigmoid(ph))
        bx0, by0, bx1, by1 = _corners(tx, ty, tw, th)
        zero = np.float32(0.0)
        iw = jnp.maximum(jnp.minimum(ax1, bx1) - jnp.maximum(ax0, bx0), zero)
        ih = jnp.maximum(jnp.minimum(ay1, by1) - jnp.maximum(ay0, by0), zero)
        inter = iw * ih
        area_a = (ax1 - ax0) * (ay1 - ay0)
        area_b = (bx1 - bx0) * (by1 - by0)
        union = area_a + area_b - inter
        iou = inter / (union + np.float32(1e-9))
        cw = jnp.maximum(jnp.maximum(ax1, bx1) - jnp.minimum(ax0, bx0), zero)
        ch = jnp.maximum(jnp.maximum(ay1, by1) - jnp.minimum(ay0, by0), zero)
        c_area = cw * ch
        giou = iou - (c_area - union) / (c_area + np.float32(1e-9))
        a_giou = a_giou + (np.float32(1.0) - giou) * conf
        return a_noobj, a_obj, a_cnt, a_giou

    z = jnp.zeros((16,), jnp.float32)
    a_noobj, a_obj, a_cnt, a_giou = lax.fori_loop(
        0, CHUNKS, chunk, (z, z, z, z))

    ob[pl.ds(0, 16)] = a_noobj
    ob[pl.ds(16, 16)] = a_obj
    ob[pl.ds(32, 16)] = a_cnt
    ob[pl.ds(48, 16)] = a_giou
    pltpu.sync_copy(ob, out_hbm.at[wid])


_mesh = plsc.VectorSubcoreMesh(core_axis_name="c", subcore_axis_name="s")

_yolo_sc = functools.partial(
    pl.kernel,
    out_type=jax.ShapeDtypeStruct((N_WORKERS, 64), jnp.float32),
    mesh=_mesh,
    compiler_params=pltpu.CompilerParams(needs_layout_passes=False),
    scratch_types=[
        pltpu.VMEM((FLOATS_PW,), jnp.float32),
        pltpu.VMEM((FLOATS_PW,), jnp.float32),
        pltpu.VMEM((64,), jnp.float32),
    ],
)(_yolo_body)


def kernel(input, target):
    parts = _yolo_sc(input.reshape(-1), target.reshape(-1))
    sums = parts.reshape(N_WORKERS, 4, 16).sum(axis=(0, 2))
    s_noobj, s_obj, n_obj, s_giou = sums[0], sums[1], sums[2], sums[3]
    n_noobj = np.float32(N_ROWS) - n_obj
    loss_noobj = s_noobj / n_noobj
    loss_obj = s_obj / n_obj
    loss_bbox = s_giou / n_obj
    return (loss_obj + loss_bbox + loss_noobj, loss_noobj, loss_bbox, loss_obj)
